# baseline (speedup 1.0000x reference)
import numpy as np
import jax
import jax.numpy as jnp
from jax.experimental import pallas as pl
from jax.experimental.pallas import tpu as pltpu

_H = 40
_W = 40
_HW = _H * _W
_C = 512
_B = 8
_M = 32
_BM = _B * _M
_PC_LO_X = -51.2
_PC_LO_Y = -51.2
_DIM_X = 102.4
_DIM_Y = 102.4
_CELL_X = np.float32(_DIM_X / _W)
_CELL_Y = np.float32(_DIM_Y / _H)
_RATIO_LO = 1.0
_RATIO_HI = 6.0

_ww, _hh = np.meshgrid(range(_W), range(_H))
_wwf = (_ww.reshape(-1).astype(np.float64) + 0.5) / _W * _DIM_X + _PC_LO_X
_hhf = (_hh.reshape(-1).astype(np.float64) + 0.5) / _H * _DIM_Y + _PC_LO_Y
_GRIDS = np.stack([_wwf, _hhf], 1).astype(np.float32)

_W_CNT = np.zeros((_BM, 2 * _BM), np.float32)
for _b in range(_B):
    _sl = slice(_b * _M, (_b + 1) * _M)
    _W_CNT[_sl, _sl] = 1.0
    _W_CNT[_sl, _BM + _b * _M:_BM + (_b + 1) * _M] = np.triu(
        np.ones((_M, _M), np.float32), 1)
_E_BCAST = np.zeros((_B, _BM), np.float32)
for _b in range(_B):
    _E_BCAST[_b, _b * _M:(_b + 1) * _M] = 1.0


_VB = 2560
_VNB = _B * _HW // _VB


def _var_body(a_ref, v_ref):
    x = a_ref[0]
    s1 = jnp.sum(x, axis=1, keepdims=True)
    s2 = jnp.sum(x * x, axis=1, keepdims=True)
    v = (s2 - s1 * s1 * (1.0 / _C)) * (1.0 / (_C - 1))
    v_ref[...] = jnp.reshape(v, (1, _VB, 1))


def _var_tc(atten_blocks):
    return pl.pallas_call(
        _var_body,
        grid=(_VNB,),
        in_specs=[
            pl.BlockSpec((1, _VB, _C), lambda b: (b, b * 0, b * 0)),
        ],
        out_specs=pl.BlockSpec((1, _VB, 1), lambda b: (b, b * 0, b * 0)),
        out_shape=jax.ShapeDtypeStruct((_VNB, _VB, 1), jnp.float32),
    )(atten_blocks)


def _loss_body(vt_ref, p_ref, gr_ref, wc_ref, e_ref, out_ref):
    p = p_ref[...]
    cx = p[0:1]
    cy = p[1:2]
    dl = p[3:4]
    dw = p[4:5]
    yaw = p[6:7]
    rl = jnp.clip(_CELL_X / dl, _RATIO_LO, _RATIO_HI)
    rw = jnp.clip(_CELL_Y / dw, _RATIO_LO, _RATIO_HI)
    hx = 0.5 * dl * rl
    hy = 0.5 * dw * rw
    sn = jnp.sin(yaw)
    cs = jnp.cos(yaw)
    xs = []
    ys = []
    for sx, sy in ((-1.0, -1.0), (-1.0, 1.0), (1.0, 1.0), (1.0, -1.0)):
        lx = sx * hx
        ly = sy * hy
        xs.append(lx * cs - ly * sn + cx)
        ys.append(lx * sn + ly * cs + cy)

    gx = gr_ref[:, 0:1]
    gy = gr_ref[:, 1:2]
    crosses = []
    for k in range(4):
        kn = (k + 1) % 4
        ex = xs[kn] - xs[k]
        ey = ys[kn] - ys[k]
        ck = ey * xs[k] - ex * ys[k]
        crosses.append(ex * gy - ey * gx + ck)
    mn = jnp.minimum(jnp.minimum(crosses[0], crosses[1]),
                     jnp.minimum(crosses[2], crosses[3]))
    mx = jnp.maximum(jnp.maximum(crosses[0], crosses[1]),
                     jnp.maximum(crosses[2], crosses[3]))
    inside = (mn >= 0.0) | (mx <= 0.0)

    d2 = (gx - cx) ** 2 + (gy - cy) ** 2
    mind = jnp.min(d2, axis=0, keepdims=True)
    cellf = jax.lax.broadcasted_iota(
        jnp.int32, (_HW, _BM), 0).astype(jnp.float32)
    cand = jnp.where(d2 == mind, cellf, float(_HW))
    mi = jnp.min(cand, axis=0, keepdims=True)
    posf = (inside | (cellf == mi)).astype(jnp.float32)

    both = jax.lax.dot(posf.astype(jnp.bfloat16),
                       wc_ref[...].astype(jnp.bfloat16),
                       preferred_element_type=jnp.float32)
    kib = both[:, :_BM]
    cgt = both[:, _BM:]
    odd = (kib - jnp.floor(kib * 0.5) * 2.0) == 1.0
    own = posf * (odd & (cgt == 0.0)).astype(jnp.float32)

    a9 = jnp.concatenate(
        [vt_ref[...], jnp.ones((1, _HW), jnp.float32)], axis=0)
    sums = jax.lax.dot(a9, own,
                       precision=jax.lax.Precision.HIGHEST)
    vs = jnp.sum(sums[:_B] * e_ref[...], axis=0, keepdims=True)
    cnt = sums[_B:_B + 1]
    has = cnt > 0.0
    contrib = jnp.where(has, vs / jnp.maximum(cnt, 1.0), 0.0)
    loss = -jnp.sum(contrib)
    posn = jnp.sum(has.astype(jnp.float32))
    out_ref[0, 0] = loss / jnp.maximum(posn, 1.0)


def _loss_tc(vb, params, grids, wc, eb):
    out = pl.pallas_call(
        _loss_body,
        out_specs=pl.BlockSpec(memory_space=pltpu.SMEM),
        out_shape=jax.ShapeDtypeStruct((1, 1), jnp.float32),
    )(vb, params, grids, wc, eb)
    return out[0, 0]


def kernel(atten_map, gt_bboxes):
    vb = _var_tc(
        atten_map.reshape(_VNB, _VB, _C)).reshape(_B, _HW)
    params = jnp.transpose(gt_bboxes.astype(jnp.float32),
                           (2, 0, 1)).reshape(7, _BM)
    grids = jnp.asarray(_GRIDS)
    return _loss_tc(vb, params, grids, jnp.asarray(_W_CNT),
                    jnp.asarray(_E_BCAST))


# speedup vs baseline: 1.0641x; 1.0641x over previous
import numpy as np
import jax
import jax.numpy as jnp
from jax.experimental import pallas as pl
from jax.experimental.pallas import tpu as pltpu

_H = 40
_W = 40
_HW = _H * _W
_C = 512
_B = 8
_M = 32
_BM = _B * _M
_PC_LO_X = -51.2
_PC_LO_Y = -51.2
_DIM_X = 102.4
_DIM_Y = 102.4
_CELL_X = np.float32(_DIM_X / _W)
_CELL_Y = np.float32(_DIM_Y / _H)
_RATIO_LO = 1.0
_RATIO_HI = 6.0

_ww, _hh = np.meshgrid(range(_W), range(_H))
_wwf = (_ww.reshape(-1).astype(np.float64) + 0.5) / _W * _DIM_X + _PC_LO_X
_hhf = (_hh.reshape(-1).astype(np.float64) + 0.5) / _H * _DIM_Y + _PC_LO_Y
_GRIDS = np.stack([_wwf, _hhf], 1).astype(np.float32)

_W_CNT = np.zeros((_BM, 2 * _BM), np.float32)
for _b in range(_B):
    _sl = slice(_b * _M, (_b + 1) * _M)
    _W_CNT[_sl, _sl] = 1.0
    _W_CNT[_sl, _BM + _b * _M:_BM + (_b + 1) * _M] = np.triu(
        np.ones((_M, _M), np.float32), 1)
_E_BCAST = np.zeros((_B, _BM), np.float32)
for _b in range(_B):
    _E_BCAST[_b, _b * _M:(_b + 1) * _M] = 1.0


_VB = 3200
_VNB = _B * _HW // _VB


def _var_body(a_ref, v_ref):
    x = a_ref[0]
    s1 = jnp.sum(x, axis=1, keepdims=True)
    s2 = jnp.sum(x * x, axis=1, keepdims=True)
    v = (s2 - s1 * s1 * (1.0 / _C)) * (1.0 / (_C - 1))
    v_ref[...] = jnp.reshape(v, (1, _VB, 1))


def _var_tc(atten_blocks):
    return pl.pallas_call(
        _var_body,
        grid=(_VNB,),
        in_specs=[
            pl.BlockSpec((1, _VB, _C), lambda b: (b, b * 0, b * 0)),
        ],
        out_specs=pl.BlockSpec((1, _VB, 1), lambda b: (b, b * 0, b * 0)),
        out_shape=jax.ShapeDtypeStruct((_VNB, _VB, 1), jnp.float32),
    )(atten_blocks)


def _loss_body(vt_ref, p_ref, gr_ref, wc_ref, e_ref, out_ref):
    p = p_ref[...]
    cx = p[0:1]
    cy = p[1:2]
    dl = p[3:4]
    dw = p[4:5]
    yaw = p[6:7]
    rl = jnp.clip(_CELL_X / dl, _RATIO_LO, _RATIO_HI)
    rw = jnp.clip(_CELL_Y / dw, _RATIO_LO, _RATIO_HI)
    hx = 0.5 * dl * rl
    hy = 0.5 * dw * rw
    sn = jnp.sin(yaw)
    cs = jnp.cos(yaw)
    xs = []
    ys = []
    for sx, sy in ((-1.0, -1.0), (-1.0, 1.0), (1.0, 1.0), (1.0, -1.0)):
        lx = sx * hx
        ly = sy * hy
        xs.append(lx * cs - ly * sn + cx)
        ys.append(lx * sn + ly * cs + cy)

    gx = gr_ref[:, 0:1]
    gy = gr_ref[:, 1:2]
    crosses = []
    for k in range(4):
        kn = (k + 1) % 4
        ex = xs[kn] - xs[k]
        ey = ys[kn] - ys[k]
        ck = ey * xs[k] - ex * ys[k]
        crosses.append(ex * gy - ey * gx + ck)
    mn = jnp.minimum(jnp.minimum(crosses[0], crosses[1]),
                     jnp.minimum(crosses[2], crosses[3]))
    mx = jnp.maximum(jnp.maximum(crosses[0], crosses[1]),
                     jnp.maximum(crosses[2], crosses[3]))
    inside = (mn >= 0.0) | (mx <= 0.0)

    d2 = (gx - cx) ** 2 + (gy - cy) ** 2
    mind = jnp.min(d2, axis=0, keepdims=True)
    cellf = jax.lax.broadcasted_iota(
        jnp.int32, (_HW, _BM), 0).astype(jnp.float32)
    cand = jnp.where(d2 == mind, cellf, float(_HW))
    mi = jnp.min(cand, axis=0, keepdims=True)
    posf = (inside | (cellf == mi)).astype(jnp.float32)

    both = jax.lax.dot(posf.astype(jnp.bfloat16),
                       wc_ref[...].astype(jnp.bfloat16),
                       preferred_element_type=jnp.float32)
    kib = both[:, :_BM]
    cgt = both[:, _BM:]
    odd = (kib - jnp.floor(kib * 0.5) * 2.0) == 1.0
    own = posf * (odd & (cgt == 0.0)).astype(jnp.float32)

    a9 = jnp.concatenate(
        [vt_ref[...], jnp.ones((1, _HW), jnp.float32)], axis=0)
    sums = jax.lax.dot(a9, own)
    vs = jnp.sum(sums[:_B] * e_ref[...], axis=0, keepdims=True)
    cnt = sums[_B:_B + 1]
    has = cnt > 0.0
    contrib = jnp.where(has, vs / jnp.maximum(cnt, 1.0), 0.0)
    loss = -jnp.sum(contrib)
    posn = jnp.sum(has.astype(jnp.float32))
    out_ref[0, 0] = loss / jnp.maximum(posn, 1.0)


def _loss_tc(vb, params, grids, wc, eb):
    out = pl.pallas_call(
        _loss_body,
        out_specs=pl.BlockSpec(memory_space=pltpu.SMEM),
        out_shape=jax.ShapeDtypeStruct((1, 1), jnp.float32),
    )(vb, params, grids, wc, eb)
    return out[0, 0]


def kernel(atten_map, gt_bboxes):
    vb = _var_tc(
        atten_map.reshape(_VNB, _VB, _C)).reshape(_B, _HW)
    params = jnp.transpose(gt_bboxes.astype(jnp.float32),
                           (2, 0, 1)).reshape(7, _BM)
    grids = jnp.asarray(_GRIDS)
    return _loss_tc(vb, params, grids, jnp.asarray(_W_CNT),
                    jnp.asarray(_E_BCAST))
